# Initial kernel scaffold; baseline (speedup 1.0000x reference)
import jax
import jax.numpy as jnp
from jax.experimental import pallas as pl


def kernel(x, w1p, b1p, w2p, b2p):
    raise NotImplementedError("write your pallas kernel here")


# speedup vs baseline: 1.1178x; 1.1178x over previous
import jax
import jax.numpy as jnp
from jax.experimental import pallas as pl
from jax.experimental.pallas import tpu as pltpu


def _round_up(n, m):
    return ((n + m - 1) // m) * m


def _mlp_packed_body(x_ref, w1_ref, b1_ref, w2_ref, b2_ref, o_ref):
    h = jnp.dot(x_ref[...], w1_ref[...], preferred_element_type=jnp.float32)
    h = jnp.maximum(h + b1_ref[...], 0.0)
    out = jnp.dot(h, w2_ref[...], preferred_element_type=jnp.float32)
    o_ref[...] = (out + b2_ref[...]).astype(o_ref.dtype)


def _block_diag(w, p):
    d, h = w.shape
    eye = jnp.eye(p, dtype=w.dtype)
    return (eye[:, None, :, None] * w[None, :, None, :]).reshape(p * d, p * h)


def kernel(x, w1p, b1p, w2p, b2p):
    B, D = x.shape
    Hp = w1p.shape[1]
    O = w2p.shape[1]
    f32 = jnp.float32
    x = x.astype(f32)

    P = 1
    while P * 2 * D <= 512 and B % (P * 2) == 0 and (P * 2) * D <= 128:
        P *= 2

    xr = x.reshape(B // P, P * D)
    w1b = _block_diag(w1p.astype(f32), P)
    w2b = _block_diag(w2p.astype(f32), P)
    b1b = jnp.tile(b1p.astype(f32), (1, P))
    b2b = jnp.tile(b2p.astype(f32), (1, P))

    Bp = B // P
    block_m = min(2048, max(_round_up(-(-Bp // 2), 8), 8))
    Bpp = _round_up(Bp, block_m)
    if Bpp != Bp:
        xr = jnp.zeros((Bpp, P * D), f32).at[:Bp].set(xr)

    out_p = pl.pallas_call(
        _mlp_packed_body,
        out_shape=jax.ShapeDtypeStruct((Bpp, P * O), f32),
        grid_spec=pl.GridSpec(
            grid=(Bpp // block_m,),
            in_specs=[
                pl.BlockSpec((block_m, P * D), lambda i: (i, 0)),
                pl.BlockSpec((P * D, P * Hp), lambda i: (0, 0)),
                pl.BlockSpec((1, P * Hp), lambda i: (0, 0)),
                pl.BlockSpec((P * Hp, P * O), lambda i: (0, 0)),
                pl.BlockSpec((1, P * O), lambda i: (0, 0)),
            ],
            out_specs=pl.BlockSpec((block_m, P * O), lambda i: (i, 0)),
        ),
        compiler_params=pltpu.CompilerParams(
            dimension_semantics=("parallel",)),
    )(xr, w1b, b1b, w2b, b2b)

    return out_p[:Bp].reshape(B, O)


# speedup vs baseline: 1.5444x; 1.3816x over previous
import jax
import jax.numpy as jnp
from jax.experimental import pallas as pl
from jax.experimental.pallas import tpu as pltpu


def _round_up(n, m):
    return ((n + m - 1) // m) * m


def _mlp_block_body(x_ref, w1_ref, b1_ref, w2_ref, b2_ref, o_ref, *, chunks):
    bm = x_ref.shape[0]
    c = bm // chunks
    for j in range(chunks):
        sl = pl.ds(j * c, c)
        h = jnp.dot(x_ref[sl, :], w1_ref[...],
                    preferred_element_type=jnp.float32)
        h = jnp.maximum(h + b1_ref[...], 0.0)
        out = jnp.dot(h, w2_ref[...], preferred_element_type=jnp.float32)
        o_ref[sl, :] = (out + b2_ref[...]).astype(o_ref.dtype)


def kernel(x, w1p, b1p, w2p, b2p):
    B, D = x.shape
    Hp = w1p.shape[1]
    O = w2p.shape[1]
    f32 = jnp.float32
    x = x.astype(f32)

    block_m = min(8192, max(_round_up(-(-B // 2), 8), 8))
    Bp = _round_up(B, block_m)
    if Bp != B:
        x = jnp.zeros((Bp, D), f32).at[:B].set(x)
    chunks = 4 if block_m % 4096 == 0 else 1

    import functools
    out_p = pl.pallas_call(
        functools.partial(_mlp_block_body, chunks=chunks),
        out_shape=jax.ShapeDtypeStruct((Bp, O), f32),
        grid_spec=pl.GridSpec(
            grid=(Bp // block_m,),
            in_specs=[
                pl.BlockSpec((block_m, D), lambda i: (i, 0)),
                pl.BlockSpec((D, Hp), lambda i: (0, 0)),
                pl.BlockSpec((1, Hp), lambda i: (0, 0)),
                pl.BlockSpec((Hp, O), lambda i: (0, 0)),
                pl.BlockSpec((1, O), lambda i: (0, 0)),
            ],
            out_specs=pl.BlockSpec((block_m, O), lambda i: (i, 0)),
        ),
        compiler_params=pltpu.CompilerParams(
            dimension_semantics=("parallel",)),
    )(x, w1p, b1p, w2p, b2p)

    return out_p[:B]
